# Initial kernel scaffold; baseline (speedup 1.0000x reference)
import jax
import jax.numpy as jnp
from jax.experimental import pallas as pl


def kernel(x, edge_index, edge_attr, batch, We, be, linW, linb, mlpW, mlpb, n2W, n2b, W1, b1, W2, b2, W3, b3):
    raise NotImplementedError("write your pallas kernel here")


# speedup vs baseline: 2.3657x; 2.3657x over previous
import functools

import jax
import jax.numpy as jnp
from jax import lax
from jax.experimental import pallas as pl
from jax.experimental.pallas import tpu as pltpu
from jax.experimental.pallas import tpu_sc as plsc

N = 10000
E = 160000
DF = 128
DE = 16
H = 300
HP = 384
G = 64
DEPTH = 3

_NC = 2
_NS = 16
_NW = _NC * _NS


def _sc_gather(table, idx):
    e_tot = idx.shape[0]
    per_w = e_tot // _NW
    ch = 40
    n_ch = per_w // ch
    mesh = plsc.VectorSubcoreMesh(core_axis_name="c", subcore_axis_name="s")

    @functools.partial(
        pl.kernel,
        out_type=jax.ShapeDtypeStruct((e_tot, HP), jnp.float32),
        mesh=mesh,
        scratch_types=[
            pltpu.VMEM((ch,), jnp.int32),
            pltpu.VMEM((ch, HP), jnp.float32),
            pltpu.SemaphoreType.DMA,
        ],
    )
    def k(table_hbm, idx_hbm, out_hbm, idx_v, rows_v, sem):
        wid = lax.axis_index("s") * _NC + lax.axis_index("c")
        base0 = wid * per_w

        def body(i, carry):
            base = base0 + i * ch
            pltpu.sync_copy(idx_hbm.at[pl.ds(base, ch)], idx_v)
            pltpu.async_copy(table_hbm.at[idx_v], rows_v, sem).wait()
            pltpu.sync_copy(rows_v, out_hbm.at[pl.ds(base, ch)])
            return carry

        lax.fori_loop(0, n_ch, body, 0)

    return k(table, idx)


def _sc_scatter_add(msg, idx):
    e_tot = msg.shape[0]
    per_c = e_tot // _NC
    per_t = per_c // _NS
    ch = 128
    n_ch = per_t // ch
    tail = per_t - n_ch * ch
    slab_rows = 10240
    rows_t = slab_rows // _NS
    zch = 32
    n_phase = HP // 128
    mesh = plsc.VectorSubcoreMesh(core_axis_name="c", subcore_axis_name="s")

    @functools.partial(
        pl.kernel,
        out_type=jax.ShapeDtypeStruct((_NC, N, HP), jnp.float32),
        mesh=mesh,
        scratch_types=[
            pltpu.VMEM_SHARED((slab_rows, 128), jnp.float32),
            pltpu.VMEM((ch,), jnp.int32),
            pltpu.VMEM((tail,), jnp.int32),
            pltpu.VMEM((ch, 128), jnp.float32),
            pltpu.VMEM((zch, 128), jnp.float32),
        ],
    )
    def k(msg_hbm, idx_hbm, out_hbm, slab, idx_v, idxt_v, buf_v, zero_v):
        c = lax.axis_index("c")
        s = lax.axis_index("s")
        base0 = c * per_c + s * per_t

        zval = jnp.zeros((16,), jnp.float32)
        for zr in range(zch):
            for zc in range(8):
                zero_v[zr, pl.ds(zc * 16, 16)] = zval

        for j in range(n_phase):
            def zcopy(i, carry):
                pltpu.sync_copy(zero_v, slab.at[pl.ds((s * (rows_t // zch) + i) * zch, zch)])
                return carry

            lax.fori_loop(0, rows_t // zch, zcopy, 0)
            plsc.subcore_barrier()

            def body(i, carry):
                base = base0 + i * ch
                pltpu.sync_copy(idx_hbm.at[pl.ds(base, ch)], idx_v)
                pltpu.sync_copy(
                    msg_hbm.at[pl.ds(base, ch), pl.ds(j * 128, 128)], buf_v)
                pltpu.sync_copy(buf_v, slab.at[idx_v], add=True)
                return carry

            lax.fori_loop(0, n_ch, body, 0)
            tbase = base0 + n_ch * ch
            pltpu.sync_copy(idx_hbm.at[pl.ds(tbase, tail)], idxt_v)
            pltpu.sync_copy(
                msg_hbm.at[pl.ds(tbase, tail), pl.ds(j * 128, 128)],
                buf_v.at[pl.ds(0, tail)])
            pltpu.sync_copy(buf_v.at[pl.ds(0, tail)], slab.at[idxt_v], add=True)
            plsc.subcore_barrier()

            wr = 640
            nfull = N // wr
            rem = N - nfull * wr

            @pl.when(s < nfull)
            def _():
                pltpu.sync_copy(
                    slab.at[pl.ds(s * wr, wr)],
                    out_hbm.at[c, pl.ds(s * wr, wr), pl.ds(j * 128, 128)])

            @pl.when(s == _NS - 1)
            def _():
                pltpu.sync_copy(
                    slab.at[pl.ds(nfull * wr, rem)],
                    out_hbm.at[c, pl.ds(nfull * wr, rem), pl.ds(j * 128, 128)])
            plsc.subcore_barrier()

    return k(msg, idx)


def _tc_matmul(xin, w, b, act, bm):
    m, kdim = xin.shape
    np_ = w.shape[1]
    grid = m // bm

    def body(x_ref, w_ref, b_ref, o_ref):
        r = jnp.dot(x_ref[...], w_ref[...], preferred_element_type=jnp.float32)
        r = r + b_ref[...]
        o_ref[...] = act(r)

    return pl.pallas_call(
        body,
        grid=(grid,),
        in_specs=[
            pl.BlockSpec((bm, kdim), lambda i: (i, 0)),
            pl.BlockSpec((kdim, np_), lambda i: (0, 0)),
            pl.BlockSpec((1, np_), lambda i: (0, 0)),
        ],
        out_specs=pl.BlockSpec((bm, np_), lambda i: (i, 0)),
        out_shape=jax.ShapeDtypeStruct((m, np_), jnp.float32),
    )(xin, w, b)


def _tc_edge_init(xg, ea, we2, be, bm):
    grid = E // bm

    def body(xg_ref, ea_ref, w_ref, b_ref, o_ref):
        r = jnp.dot(ea_ref[...], w_ref[...], preferred_element_type=jnp.float32)
        r = r + xg_ref[...] + b_ref[...]
        o_ref[...] = jax.nn.silu(r)

    return pl.pallas_call(
        body,
        grid=(grid,),
        in_specs=[
            pl.BlockSpec((bm, HP), lambda i: (i, 0)),
            pl.BlockSpec((bm, DE), lambda i: (i, 0)),
            pl.BlockSpec((DE, HP), lambda i: (0, 0)),
            pl.BlockSpec((1, HP), lambda i: (0, 0)),
        ],
        out_specs=pl.BlockSpec((bm, HP), lambda i: (i, 0)),
        out_shape=jax.ShapeDtypeStruct((E, HP), jnp.float32),
    )(xg, ea, we2, be)


def _tc_edge_update(ag2, e2, w, b, last, bm2):
    e2_tot = E // 2
    grid = e2_tot // bm2

    def body(ag_ref, e_ref, w_ref, b_ref, o_ref):
        u_l = ag_ref[:, :HP] - e_ref[:, HP:]
        u_r = ag_ref[:, HP:] - e_ref[:, :HP]
        h_l = jnp.dot(u_l, w_ref[...], preferred_element_type=jnp.float32)
        h_r = jnp.dot(u_r, w_ref[...], preferred_element_type=jnp.float32)
        h_l = jnp.maximum(h_l + b_ref[...], 0.0)
        h_r = jnp.maximum(h_r + b_ref[...], 0.0)
        if last:
            o_l = h_l + h_l
            o_r = h_r + h_r
        else:
            o_l = jax.nn.silu(h_l) + h_l
            o_r = jax.nn.silu(h_r) + h_r
        o_ref[:, :HP] = o_l
        o_ref[:, HP:] = o_r

    return pl.pallas_call(
        body,
        grid=(grid,),
        in_specs=[
            pl.BlockSpec((bm2, 2 * HP), lambda i: (i, 0)),
            pl.BlockSpec((bm2, 2 * HP), lambda i: (i, 0)),
            pl.BlockSpec((HP, HP), lambda i: (0, 0)),
            pl.BlockSpec((1, HP), lambda i: (0, 0)),
        ],
        out_specs=pl.BlockSpec((bm2, 2 * HP), lambda i: (i, 0)),
        out_shape=jax.ShapeDtypeStruct((e2_tot, 2 * HP), jnp.float32),
    )(ag2, e2, w, b)


def _tc_combine(p):
    bm = 2000
    grid = N // bm

    def body(p_ref, o_ref):
        o_ref[...] = p_ref[0] + p_ref[1]

    return pl.pallas_call(
        body,
        grid=(grid,),
        in_specs=[pl.BlockSpec((2, bm, HP), lambda i: (0, i, 0))],
        out_specs=pl.BlockSpec((bm, HP), lambda i: (i, 0)),
        out_shape=jax.ShapeDtypeStruct((N, HP), jnp.float32),
    )(p)


def _tc_pool_ffn(node_p, batch2, w1, b1, w2, b2, w3, b3, bn):
    grid = N // bn

    def body(nh_ref, bt_ref, w1_ref, b1_ref, w2_ref, b2_ref, w3_ref, b3_ref,
             o_ref, acc):
        pid = pl.program_id(0)

        @pl.when(pid == 0)
        def _():
            acc[...] = jnp.zeros_like(acc)

        oh = (bt_ref[...] == lax.broadcasted_iota(jnp.int32, (bn, G), 1))
        oh = oh.astype(jnp.float32)
        nh = nh_ref[0] + nh_ref[1]
        acc[...] += lax.dot_general(
            oh, nh,
            dimension_numbers=(((0,), (0,)), ((), ())),
            preferred_element_type=jnp.float32)

        @pl.when(pid == grid - 1)
        def _():
            p = acc[...]
            h = p @ w1_ref[...] + b1_ref[...]
            h = jax.nn.silu(h)
            h = h @ w2_ref[...] + b2_ref[...]
            h = jax.nn.silu(h)
            o_ref[...] = h @ w3_ref[...] + b3_ref[...]

    return pl.pallas_call(
        body,
        grid=(grid,),
        in_specs=[
            pl.BlockSpec((2, bn, HP), lambda i: (0, i, 0)),
            pl.BlockSpec((bn, 1), lambda i: (i, 0)),
            pl.BlockSpec((HP, H), lambda i: (0, 0)),
            pl.BlockSpec((1, H), lambda i: (0, 0)),
            pl.BlockSpec((H, H), lambda i: (0, 0)),
            pl.BlockSpec((1, H), lambda i: (0, 0)),
            pl.BlockSpec((H, 1), lambda i: (0, 0)),
            pl.BlockSpec((1, 1), lambda i: (0, 0)),
        ],
        out_specs=pl.BlockSpec((G, 1), lambda i: (0, 0)),
        out_shape=jax.ShapeDtypeStruct((G, 1), jnp.float32),
        scratch_shapes=[pltpu.VMEM((G, HP), jnp.float32)],
    )(node_p, batch2, w1, b1, w2, b2, w3, b3)


def _padc(w, cols):
    return jnp.pad(w, ((0, 0), (0, cols - w.shape[1])))


def kernel(x, edge_index, edge_attr, batch, We, be, linW, linb, mlpW, mlpb,
           n2W, n2b, W1, b1, W2, b2, W3, b3):
    row = edge_index[0].astype(jnp.int32)
    col = edge_index[1].astype(jnp.int32)

    weX = _padc(We[:DF], HP)
    weE = _padc(We[DF:], HP)
    beP = _padc(be[None, :], HP)
    linWP = jnp.pad(linW, ((0, 0), (0, HP - H), (0, HP - H)))
    linbP = jnp.pad(linb, ((0, 0), (0, HP - H)))
    mlpWP = jnp.pad(mlpW, ((0, 0), (0, HP - H), (0, HP - H)))
    mlpbP = jnp.pad(mlpb, ((0, 0), (0, HP - H)))
    n2WP = jnp.pad(n2W, ((0, HP - H), (0, HP - H)))
    n2bP = _padc(n2b[None, :], HP)
    w1P = jnp.pad(W1, ((0, HP - H), (0, 0)))

    xw = _tc_matmul(x, weX, jnp.zeros((1, HP), jnp.float32),
                    lambda v: v, 2000)
    xg = _sc_gather(xw, row)
    e = _tc_edge_init(xg, edge_attr, weE, beP, 1000)

    relu = lambda v: jnp.maximum(v, 0.0)
    for l in range(DEPTH):
        msg = _tc_matmul(e, linWP[l], linbP[l][None, :], relu, 1000)
        a = _tc_combine(_sc_scatter_add(msg, col))
        ag = _sc_gather(a, row)
        e = _tc_edge_update(
            ag.reshape(E // 2, 2 * HP), e.reshape(E // 2, 2 * HP),
            mlpWP[l], mlpbP[l][None, :], l == DEPTH - 1, 800,
        ).reshape(E, HP)

    msg_f = _tc_matmul(e, n2WP, n2bP, relu, 1000)
    node_p = _sc_scatter_add(msg_f, col)

    batch2 = batch.astype(jnp.int32)[:, None]
    return _tc_pool_ffn(node_p, batch2, w1P, b1[None, :], W2, b2[None, :],
                        W3, b3[None, :], 2000)


# speedup vs baseline: 2.8332x; 1.1976x over previous
import functools

import jax
import jax.numpy as jnp
from jax import lax
from jax.experimental import pallas as pl
from jax.experimental.pallas import tpu as pltpu
from jax.experimental.pallas import tpu_sc as plsc

N = 10000
E = 160000
DF = 128
DE = 16
H = 300
HP = 384
G = 64
DEPTH = 3

_NC = 2
_NS = 16
_NW = _NC * _NS


def _sc_gather(table, idx):
    e_tot = idx.shape[0]
    per_w = e_tot // _NW
    ch = 128
    n_ch = per_w // ch
    tail = per_w - n_ch * ch
    mesh = plsc.VectorSubcoreMesh(core_axis_name="c", subcore_axis_name="s")

    @functools.partial(
        pl.kernel,
        out_type=jax.ShapeDtypeStruct((e_tot, HP), jnp.float32),
        mesh=mesh,
        scratch_types=[
            pltpu.VMEM((2, ch), jnp.int32),
            pltpu.VMEM((tail,), jnp.int32),
            pltpu.VMEM((2, ch, HP), jnp.float32),
            pltpu.VMEM((tail, HP), jnp.float32),
        ] + [pltpu.SemaphoreType.DMA] * 7,
    )
    def k(table_hbm, idx_hbm, out_hbm, idx_v, idxt_v, rows_v, rowst_v, *sems):
        ixs, gs, ws, ts = sems[0:2], sems[2:4], sems[4:6], sems[6]
        wid = lax.axis_index("s") * _NC + lax.axis_index("c")
        base0 = wid * per_w

        def start_idx(i):
            return pltpu.async_copy(
                idx_hbm.at[pl.ds(base0 + i * ch, ch)], idx_v.at[i % 2],
                ixs[i % 2])

        def start_gather(i):
            return pltpu.async_copy(
                table_hbm.at[idx_v.at[i % 2]], rows_v.at[i % 2], gs[i % 2])

        def start_write(i):
            return pltpu.async_copy(
                rows_v.at[i % 2], out_hbm.at[pl.ds(base0 + i * ch, ch)],
                ws[i % 2])

        tb = base0 + n_ch * ch
        pltpu.async_copy(idx_hbm.at[pl.ds(tb, tail)], idxt_v, ts).wait()
        pltpu.async_copy(table_hbm.at[idxt_v], rowst_v, ts).wait()
        th = pltpu.async_copy(rowst_v, out_hbm.at[pl.ds(tb, tail)], ts)

        ih, gh, wh = {}, {}, {}
        ih[0] = start_idx(0)
        if n_ch > 1:
            ih[1] = start_idx(1)
        ih[0].wait()
        gh[0] = start_gather(0)
        for i in range(n_ch):
            gh[i].wait()
            wh[i] = start_write(i)
            if i + 2 < n_ch:
                ih[i + 2] = start_idx(i + 2)
            if i + 1 < n_ch:
                if i - 1 >= 0:
                    wh[i - 1].wait()
                ih[i + 1].wait()
                gh[i + 1] = start_gather(i + 1)
        if n_ch > 1:
            wh[n_ch - 2].wait()
        wh[n_ch - 1].wait()
        th.wait()

    return k(table, idx)


def _sc_scatter_add(msg, idx):
    e_tot = msg.shape[0]
    per_c = e_tot // _NC
    per_t = per_c // _NS
    ch = 72
    n_ch = per_t // ch
    tail = per_t - n_ch * ch
    slab_rows = 10240
    rows_t = slab_rows // _NS
    zch = 16
    n_phase = HP // 128
    nslot = 4
    mesh = plsc.VectorSubcoreMesh(core_axis_name="c", subcore_axis_name="s")

    @functools.partial(
        pl.kernel,
        out_type=jax.ShapeDtypeStruct((_NC, N, HP), jnp.float32),
        mesh=mesh,
        scratch_types=[
            pltpu.VMEM_SHARED((slab_rows, 128), jnp.float32),
            pltpu.VMEM((nslot, ch), jnp.int32),
            pltpu.VMEM((tail,), jnp.int32),
            pltpu.VMEM((nslot, ch, 128), jnp.float32),
            pltpu.VMEM((tail, 128), jnp.float32),
            pltpu.VMEM((zch, 128), jnp.float32),
        ] + [pltpu.SemaphoreType.DMA] * (3 * nslot + 1),
    )
    def k(msg_hbm, idx_hbm, out_hbm, slab, idx_v, idxt_v, buf_v, buft_v,
          zero_v, *sems):
        ixs, ms, sas, ts = (sems[0:nslot], sems[nslot:2 * nslot],
                            sems[2 * nslot:3 * nslot], sems[3 * nslot])
        c = lax.axis_index("c")
        s = lax.axis_index("s")
        base0 = c * per_c + s * per_t

        zval = jnp.zeros((16,), jnp.float32)
        for zr in range(zch):
            for zc in range(8):
                zero_v[zr, pl.ds(zc * 16, 16)] = zval

        for j in range(n_phase):
            def zcopy(i, carry):
                pltpu.sync_copy(
                    zero_v,
                    slab.at[pl.ds((s * (rows_t // zch) + i) * zch, zch)])
                return carry

            lax.fori_loop(0, rows_t // zch, zcopy, 0)
            plsc.subcore_barrier()

            def start_loads(i):
                p = i % nslot
                a = pltpu.async_copy(
                    idx_hbm.at[pl.ds(base0 + i * ch, ch)], idx_v.at[p], ixs[p])
                b = pltpu.async_copy(
                    msg_hbm.at[pl.ds(base0 + i * ch, ch), pl.ds(j * 128, 128)],
                    buf_v.at[p], ms[p])
                return a, b

            def start_sa(i):
                p = i % nslot
                return pltpu.async_copy(buf_v.at[p], slab.at[idx_v.at[p]],
                                        sas[p], add=True)

            tb = base0 + n_ch * ch
            pltpu.async_copy(idx_hbm.at[pl.ds(tb, tail)], idxt_v, ts).wait()
            pltpu.async_copy(
                msg_hbm.at[pl.ds(tb, tail), pl.ds(j * 128, 128)],
                buft_v, ts).wait()
            th = pltpu.async_copy(buft_v, slab.at[idxt_v], ts, add=True)

            lh, sh = {}, {}
            waited = set()
            for i in range(min(nslot - 1, n_ch)):
                lh[i] = start_loads(i)
            for i in range(n_ch):
                lh[i][0].wait()
                lh[i][1].wait()
                sh[i] = start_sa(i)
                if i + nslot - 1 < n_ch:
                    if i - 1 >= 0:
                        sh[i - 1].wait()
                        waited.add(i - 1)
                    lh[i + nslot - 1] = start_loads(i + nslot - 1)
            for i in range(n_ch):
                if i not in waited:
                    sh[i].wait()
            th.wait()
            plsc.subcore_barrier()

            wr = 640
            nfull = N // wr
            rem = N - nfull * wr

            @pl.when(s < nfull)
            def _():
                pltpu.sync_copy(
                    slab.at[pl.ds(s * wr, wr)],
                    out_hbm.at[c, pl.ds(s * wr, wr), pl.ds(j * 128, 128)])

            @pl.when(s == _NS - 1)
            def _():
                pltpu.sync_copy(
                    slab.at[pl.ds(nfull * wr, rem)],
                    out_hbm.at[c, pl.ds(nfull * wr, rem), pl.ds(j * 128, 128)])
            plsc.subcore_barrier()

    return k(msg, idx)


def _tc_matmul(xin, w, b, act, bm):
    m, kdim = xin.shape
    np_ = w.shape[1]
    grid = m // bm

    def body(x_ref, w_ref, b_ref, o_ref):
        r = jnp.dot(x_ref[...], w_ref[...], preferred_element_type=jnp.float32)
        r = r + b_ref[...]
        o_ref[...] = act(r)

    return pl.pallas_call(
        body,
        grid=(grid,),
        in_specs=[
            pl.BlockSpec((bm, kdim), lambda i: (i, 0)),
            pl.BlockSpec((kdim, np_), lambda i: (0, 0)),
            pl.BlockSpec((1, np_), lambda i: (0, 0)),
        ],
        out_specs=pl.BlockSpec((bm, np_), lambda i: (i, 0)),
        out_shape=jax.ShapeDtypeStruct((m, np_), jnp.float32),
    )(xin, w, b)


def _tc_edge_init(xg, ea, we2, be, bm):
    grid = E // bm

    def body(xg_ref, ea_ref, w_ref, b_ref, o_ref):
        r = jnp.dot(ea_ref[...], w_ref[...], preferred_element_type=jnp.float32)
        r = r + xg_ref[...] + b_ref[...]
        o_ref[...] = jax.nn.silu(r)

    return pl.pallas_call(
        body,
        grid=(grid,),
        in_specs=[
            pl.BlockSpec((bm, HP), lambda i: (i, 0)),
            pl.BlockSpec((bm, DE), lambda i: (i, 0)),
            pl.BlockSpec((DE, HP), lambda i: (0, 0)),
            pl.BlockSpec((1, HP), lambda i: (0, 0)),
        ],
        out_specs=pl.BlockSpec((bm, HP), lambda i: (i, 0)),
        out_shape=jax.ShapeDtypeStruct((E, HP), jnp.float32),
    )(xg, ea, we2, be)


def _tc_edge_update(ag2, e2, w, b, last, bm2):
    e2_tot = E // 2
    grid = e2_tot // bm2

    def body(ag_ref, e_ref, w_ref, b_ref, o_ref):
        u_l = ag_ref[:, :HP] - e_ref[:, HP:]
        u_r = ag_ref[:, HP:] - e_ref[:, :HP]
        h_l = jnp.dot(u_l, w_ref[...], preferred_element_type=jnp.float32)
        h_r = jnp.dot(u_r, w_ref[...], preferred_element_type=jnp.float32)
        h_l = jnp.maximum(h_l + b_ref[...], 0.0)
        h_r = jnp.maximum(h_r + b_ref[...], 0.0)
        if last:
            o_l = h_l + h_l
            o_r = h_r + h_r
        else:
            o_l = jax.nn.silu(h_l) + h_l
            o_r = jax.nn.silu(h_r) + h_r
        o_ref[:, :HP] = o_l
        o_ref[:, HP:] = o_r

    return pl.pallas_call(
        body,
        grid=(grid,),
        in_specs=[
            pl.BlockSpec((bm2, 2 * HP), lambda i: (i, 0)),
            pl.BlockSpec((bm2, 2 * HP), lambda i: (i, 0)),
            pl.BlockSpec((HP, HP), lambda i: (0, 0)),
            pl.BlockSpec((1, HP), lambda i: (0, 0)),
        ],
        out_specs=pl.BlockSpec((bm2, 2 * HP), lambda i: (i, 0)),
        out_shape=jax.ShapeDtypeStruct((e2_tot, 2 * HP), jnp.float32),
    )(ag2, e2, w, b)


def _tc_combine(p):
    bm = 2000
    grid = N // bm

    def body(p_ref, o_ref):
        o_ref[...] = p_ref[0] + p_ref[1]

    return pl.pallas_call(
        body,
        grid=(grid,),
        in_specs=[pl.BlockSpec((2, bm, HP), lambda i: (0, i, 0))],
        out_specs=pl.BlockSpec((bm, HP), lambda i: (i, 0)),
        out_shape=jax.ShapeDtypeStruct((N, HP), jnp.float32),
    )(p)


def _tc_pool_ffn(node_p, batch2, w1, b1, w2, b2, w3, b3, bn):
    grid = N // bn

    def body(nh_ref, bt_ref, w1_ref, b1_ref, w2_ref, b2_ref, w3_ref, b3_ref,
             o_ref, acc):
        pid = pl.program_id(0)

        @pl.when(pid == 0)
        def _():
            acc[...] = jnp.zeros_like(acc)

        oh = (bt_ref[...] == lax.broadcasted_iota(jnp.int32, (bn, G), 1))
        oh = oh.astype(jnp.float32)
        nh = nh_ref[0] + nh_ref[1]
        acc[...] += lax.dot_general(
            oh, nh,
            dimension_numbers=(((0,), (0,)), ((), ())),
            preferred_element_type=jnp.float32)

        @pl.when(pid == grid - 1)
        def _():
            p = acc[...]
            h = p @ w1_ref[...] + b1_ref[...]
            h = jax.nn.silu(h)
            h = h @ w2_ref[...] + b2_ref[...]
            h = jax.nn.silu(h)
            o_ref[...] = h @ w3_ref[...] + b3_ref[...]

    return pl.pallas_call(
        body,
        grid=(grid,),
        in_specs=[
            pl.BlockSpec((2, bn, HP), lambda i: (0, i, 0)),
            pl.BlockSpec((bn, 1), lambda i: (i, 0)),
            pl.BlockSpec((HP, H), lambda i: (0, 0)),
            pl.BlockSpec((1, H), lambda i: (0, 0)),
            pl.BlockSpec((H, H), lambda i: (0, 0)),
            pl.BlockSpec((1, H), lambda i: (0, 0)),
            pl.BlockSpec((H, 1), lambda i: (0, 0)),
            pl.BlockSpec((1, 1), lambda i: (0, 0)),
        ],
        out_specs=pl.BlockSpec((G, 1), lambda i: (0, 0)),
        out_shape=jax.ShapeDtypeStruct((G, 1), jnp.float32),
        scratch_shapes=[pltpu.VMEM((G, HP), jnp.float32)],
    )(node_p, batch2, w1, b1, w2, b2, w3, b3)


def _padc(w, cols):
    return jnp.pad(w, ((0, 0), (0, cols - w.shape[1])))


def kernel(x, edge_index, edge_attr, batch, We, be, linW, linb, mlpW, mlpb,
           n2W, n2b, W1, b1, W2, b2, W3, b3):
    row = edge_index[0].astype(jnp.int32)
    col = edge_index[1].astype(jnp.int32)

    weX = _padc(We[:DF], HP)
    weE = _padc(We[DF:], HP)
    beP = _padc(be[None, :], HP)
    linWP = jnp.pad(linW, ((0, 0), (0, HP - H), (0, HP - H)))
    linbP = jnp.pad(linb, ((0, 0), (0, HP - H)))
    mlpWP = jnp.pad(mlpW, ((0, 0), (0, HP - H), (0, HP - H)))
    mlpbP = jnp.pad(mlpb, ((0, 0), (0, HP - H)))
    n2WP = jnp.pad(n2W, ((0, HP - H), (0, HP - H)))
    n2bP = _padc(n2b[None, :], HP)
    w1P = jnp.pad(W1, ((0, HP - H), (0, 0)))

    xw = _tc_matmul(x, weX, jnp.zeros((1, HP), jnp.float32),
                    lambda v: v, 2000)
    xg = _sc_gather(xw, row)
    e = _tc_edge_init(xg, edge_attr, weE, beP, 1000)

    relu = lambda v: jnp.maximum(v, 0.0)
    for l in range(DEPTH):
        msg = _tc_matmul(e, linWP[l], linbP[l][None, :], relu, 1000)
        a = _tc_combine(_sc_scatter_add(msg, col))
        ag = _sc_gather(a, row)
        e = _tc_edge_update(
            ag.reshape(E // 2, 2 * HP), e.reshape(E // 2, 2 * HP),
            mlpWP[l], mlpbP[l][None, :], l == DEPTH - 1, 800,
        ).reshape(E, HP)

    msg_f = _tc_matmul(e, n2WP, n2bP, relu, 1000)
    node_p = _sc_scatter_add(msg_f, col)

    batch2 = batch.astype(jnp.int32)[:, None]
    return _tc_pool_ffn(node_p, batch2, w1P, b1[None, :], W2, b2[None, :],
                        W3, b3[None, :], 2000)


# speedup vs baseline: 3.1359x; 1.1069x over previous
import functools

import jax
import jax.numpy as jnp
from jax import lax
from jax.experimental import pallas as pl
from jax.experimental.pallas import tpu as pltpu
from jax.experimental.pallas import tpu_sc as plsc

N = 10000
E = 160000
DF = 128
DE = 16
H = 300
HP = 384
G = 64
DEPTH = 3

_NC = 2
_NS = 16
_NW = _NC * _NS


def _sc_gather(table, idx):
    e_tot = idx.shape[0]
    per_w = e_tot // _NW
    ch = 128
    n_ch = per_w // ch
    tail = per_w - n_ch * ch
    mesh = plsc.VectorSubcoreMesh(core_axis_name="c", subcore_axis_name="s")

    @functools.partial(
        pl.kernel,
        out_type=jax.ShapeDtypeStruct((e_tot, HP), jnp.float32),
        mesh=mesh,
        scratch_types=[
            pltpu.VMEM((2, ch), jnp.int32),
            pltpu.VMEM((tail,), jnp.int32),
            pltpu.VMEM((2, ch, HP), jnp.float32),
            pltpu.VMEM((tail, HP), jnp.float32),
        ] + [pltpu.SemaphoreType.DMA] * 7,
    )
    def k(table_hbm, idx_hbm, out_hbm, idx_v, idxt_v, rows_v, rowst_v, *sems):
        ixs, gs, ws, ts = sems[0:2], sems[2:4], sems[4:6], sems[6]
        wid = lax.axis_index("s") * _NC + lax.axis_index("c")
        base0 = wid * per_w

        def start_idx(i):
            return pltpu.async_copy(
                idx_hbm.at[pl.ds(base0 + i * ch, ch)], idx_v.at[i % 2],
                ixs[i % 2])

        def start_gather(i):
            return pltpu.async_copy(
                table_hbm.at[idx_v.at[i % 2]], rows_v.at[i % 2], gs[i % 2])

        def start_write(i):
            return pltpu.async_copy(
                rows_v.at[i % 2], out_hbm.at[pl.ds(base0 + i * ch, ch)],
                ws[i % 2])

        tb = base0 + n_ch * ch
        pltpu.async_copy(idx_hbm.at[pl.ds(tb, tail)], idxt_v, ts).wait()
        pltpu.async_copy(table_hbm.at[idxt_v], rowst_v, ts).wait()
        th = pltpu.async_copy(rowst_v, out_hbm.at[pl.ds(tb, tail)], ts)

        ih, gh, wh = {}, {}, {}
        ih[0] = start_idx(0)
        if n_ch > 1:
            ih[1] = start_idx(1)
        ih[0].wait()
        gh[0] = start_gather(0)
        for i in range(n_ch):
            gh[i].wait()
            wh[i] = start_write(i)
            if i + 2 < n_ch:
                ih[i + 2] = start_idx(i + 2)
            if i + 1 < n_ch:
                if i - 1 >= 0:
                    wh[i - 1].wait()
                ih[i + 1].wait()
                gh[i + 1] = start_gather(i + 1)
        if n_ch > 1:
            wh[n_ch - 2].wait()
        wh[n_ch - 1].wait()
        th.wait()

    return k(table, idx)


def _sc_scatter_add(msg, idx):
    e_tot = msg.shape[0]
    per_c = e_tot // _NC
    per_t = per_c // _NS
    ch = 72
    n_ch = per_t // ch
    tail = per_t - n_ch * ch
    slab_rows = 10240
    rows_t = slab_rows // _NS
    zch = 16
    n_phase = HP // 128
    nslot = 4
    mesh = plsc.VectorSubcoreMesh(core_axis_name="c", subcore_axis_name="s")

    @functools.partial(
        pl.kernel,
        out_type=jax.ShapeDtypeStruct((_NC, N, HP), jnp.float32),
        mesh=mesh,
        scratch_types=[
            pltpu.VMEM_SHARED((slab_rows, 128), jnp.float32),
            pltpu.VMEM((nslot, ch), jnp.int32),
            pltpu.VMEM((tail,), jnp.int32),
            pltpu.VMEM((nslot, ch, 128), jnp.float32),
            pltpu.VMEM((tail, 128), jnp.float32),
            pltpu.VMEM((zch, 128), jnp.float32),
        ] + [pltpu.SemaphoreType.DMA] * (3 * nslot + 1),
    )
    def k(msg_hbm, idx_hbm, out_hbm, slab, idx_v, idxt_v, buf_v, buft_v,
          zero_v, *sems):
        ixs, ms, sas, ts = (sems[0:nslot], sems[nslot:2 * nslot],
                            sems[2 * nslot:3 * nslot], sems[3 * nslot])
        c = lax.axis_index("c")
        s = lax.axis_index("s")
        base0 = c * per_c + s * per_t

        zval = jnp.zeros((16,), jnp.float32)
        for zr in range(zch):
            for zc in range(8):
                zero_v[zr, pl.ds(zc * 16, 16)] = zval

        for j in range(n_phase):
            def zcopy(i, carry):
                pltpu.sync_copy(
                    zero_v,
                    slab.at[pl.ds((s * (rows_t // zch) + i) * zch, zch)])
                return carry

            lax.fori_loop(0, rows_t // zch, zcopy, 0)
            plsc.subcore_barrier()

            def start_loads(i):
                p = i % nslot
                a = pltpu.async_copy(
                    idx_hbm.at[pl.ds(base0 + i * ch, ch)], idx_v.at[p], ixs[p])
                b = pltpu.async_copy(
                    msg_hbm.at[pl.ds(base0 + i * ch, ch), pl.ds(j * 128, 128)],
                    buf_v.at[p], ms[p])
                return a, b

            def start_sa(i):
                p = i % nslot
                return pltpu.async_copy(buf_v.at[p], slab.at[idx_v.at[p]],
                                        sas[p], add=True)

            tb = base0 + n_ch * ch
            pltpu.async_copy(idx_hbm.at[pl.ds(tb, tail)], idxt_v, ts).wait()
            pltpu.async_copy(
                msg_hbm.at[pl.ds(tb, tail), pl.ds(j * 128, 128)],
                buft_v, ts).wait()
            th = pltpu.async_copy(buft_v, slab.at[idxt_v], ts, add=True)

            lh, sh = {}, {}
            waited = set()
            for i in range(min(nslot - 1, n_ch)):
                lh[i] = start_loads(i)
            for i in range(n_ch):
                lh[i][0].wait()
                lh[i][1].wait()
                sh[i] = start_sa(i)
                if i + nslot - 1 < n_ch:
                    if i - 1 >= 0:
                        sh[i - 1].wait()
                        waited.add(i - 1)
                    lh[i + nslot - 1] = start_loads(i + nslot - 1)
            for i in range(n_ch):
                if i not in waited:
                    sh[i].wait()
            th.wait()
            plsc.subcore_barrier()

            wr = 640
            nfull = N // wr
            rem = N - nfull * wr

            @pl.when(s < nfull)
            def _():
                pltpu.sync_copy(
                    slab.at[pl.ds(s * wr, wr)],
                    out_hbm.at[c, pl.ds(s * wr, wr), pl.ds(j * 128, 128)])

            @pl.when(s == _NS - 1)
            def _():
                pltpu.sync_copy(
                    slab.at[pl.ds(nfull * wr, rem)],
                    out_hbm.at[c, pl.ds(nfull * wr, rem), pl.ds(j * 128, 128)])
            plsc.subcore_barrier()

    return k(msg, idx)


def _tc_matmul(xin, w, b, act, bm):
    m, kdim = xin.shape
    np_ = w.shape[1]
    grid = m // bm

    def body(x_ref, w_ref, b_ref, o_ref):
        r = jnp.dot(x_ref[...].astype(jnp.float32), w_ref[...],
                    preferred_element_type=jnp.float32)
        r = r + b_ref[...]
        o_ref[...] = act(r)

    return pl.pallas_call(
        body,
        grid=(grid,),
        in_specs=[
            pl.BlockSpec((bm, kdim), lambda i: (i, 0)),
            pl.BlockSpec((kdim, np_), lambda i: (0, 0)),
            pl.BlockSpec((1, np_), lambda i: (0, 0)),
        ],
        out_specs=pl.BlockSpec((bm, np_), lambda i: (i, 0)),
        out_shape=jax.ShapeDtypeStruct((m, np_), jnp.float32),
    )(xin, w, b)


def _tc_edge_init(xg, ea, we2, be, bm):
    grid = E // bm

    def body(xg_ref, ea_ref, w_ref, b_ref, o_ref):
        r = jnp.dot(ea_ref[...], w_ref[...], preferred_element_type=jnp.float32)
        r = r + xg_ref[...] + b_ref[...]
        o_ref[...] = jax.nn.silu(r).astype(jnp.bfloat16)

    return pl.pallas_call(
        body,
        grid=(grid,),
        in_specs=[
            pl.BlockSpec((bm, HP), lambda i: (i, 0)),
            pl.BlockSpec((bm, DE), lambda i: (i, 0)),
            pl.BlockSpec((DE, HP), lambda i: (0, 0)),
            pl.BlockSpec((1, HP), lambda i: (0, 0)),
        ],
        out_specs=pl.BlockSpec((bm, HP), lambda i: (i, 0)),
        out_shape=jax.ShapeDtypeStruct((E, HP), jnp.bfloat16),
    )(xg, ea, we2, be)


def _tc_edge_update(ag2, e2, w, b, last, bm2):
    e2_tot = E // 2
    grid = e2_tot // bm2

    def body(ag_ref, e_ref, w_ref, b_ref, o_ref):
        ee = e_ref[...].astype(jnp.float32)
        u_l = ag_ref[:, :HP] - ee[:, HP:]
        u_r = ag_ref[:, HP:] - ee[:, :HP]
        h_l = jnp.dot(u_l, w_ref[...], preferred_element_type=jnp.float32)
        h_r = jnp.dot(u_r, w_ref[...], preferred_element_type=jnp.float32)
        h_l = jnp.maximum(h_l + b_ref[...], 0.0)
        h_r = jnp.maximum(h_r + b_ref[...], 0.0)
        if last:
            o_l = h_l + h_l
            o_r = h_r + h_r
        else:
            o_l = jax.nn.silu(h_l) + h_l
            o_r = jax.nn.silu(h_r) + h_r
        o_ref[:, :HP] = o_l.astype(jnp.bfloat16)
        o_ref[:, HP:] = o_r.astype(jnp.bfloat16)

    return pl.pallas_call(
        body,
        grid=(grid,),
        in_specs=[
            pl.BlockSpec((bm2, 2 * HP), lambda i: (i, 0)),
            pl.BlockSpec((bm2, 2 * HP), lambda i: (i, 0)),
            pl.BlockSpec((HP, HP), lambda i: (0, 0)),
            pl.BlockSpec((1, HP), lambda i: (0, 0)),
        ],
        out_specs=pl.BlockSpec((bm2, 2 * HP), lambda i: (i, 0)),
        out_shape=jax.ShapeDtypeStruct((e2_tot, 2 * HP), jnp.bfloat16),
    )(ag2, e2, w, b)


def _tc_combine(p):
    bm = 2000
    grid = N // bm

    def body(p_ref, o_ref):
        o_ref[...] = p_ref[0] + p_ref[1]

    return pl.pallas_call(
        body,
        grid=(grid,),
        in_specs=[pl.BlockSpec((2, bm, HP), lambda i: (0, i, 0))],
        out_specs=pl.BlockSpec((bm, HP), lambda i: (i, 0)),
        out_shape=jax.ShapeDtypeStruct((N, HP), jnp.float32),
    )(p)


def _tc_pool_ffn(node_p, batch2, w1, b1, w2, b2, w3, b3, bn):
    grid = N // bn

    def body(nh_ref, bt_ref, w1_ref, b1_ref, w2_ref, b2_ref, w3_ref, b3_ref,
             o_ref, acc):
        pid = pl.program_id(0)

        @pl.when(pid == 0)
        def _():
            acc[...] = jnp.zeros_like(acc)

        oh = (bt_ref[...] == lax.broadcasted_iota(jnp.int32, (bn, G), 1))
        oh = oh.astype(jnp.float32)
        nh = nh_ref[0] + nh_ref[1]
        acc[...] += lax.dot_general(
            oh, nh,
            dimension_numbers=(((0,), (0,)), ((), ())),
            preferred_element_type=jnp.float32)

        @pl.when(pid == grid - 1)
        def _():
            p = acc[...]
            h = p @ w1_ref[...] + b1_ref[...]
            h = jax.nn.silu(h)
            h = h @ w2_ref[...] + b2_ref[...]
            h = jax.nn.silu(h)
            o_ref[...] = h @ w3_ref[...] + b3_ref[...]

    return pl.pallas_call(
        body,
        grid=(grid,),
        in_specs=[
            pl.BlockSpec((2, bn, HP), lambda i: (0, i, 0)),
            pl.BlockSpec((bn, 1), lambda i: (i, 0)),
            pl.BlockSpec((HP, H), lambda i: (0, 0)),
            pl.BlockSpec((1, H), lambda i: (0, 0)),
            pl.BlockSpec((H, H), lambda i: (0, 0)),
            pl.BlockSpec((1, H), lambda i: (0, 0)),
            pl.BlockSpec((H, 1), lambda i: (0, 0)),
            pl.BlockSpec((1, 1), lambda i: (0, 0)),
        ],
        out_specs=pl.BlockSpec((G, 1), lambda i: (0, 0)),
        out_shape=jax.ShapeDtypeStruct((G, 1), jnp.float32),
        scratch_shapes=[pltpu.VMEM((G, HP), jnp.float32)],
    )(node_p, batch2, w1, b1, w2, b2, w3, b3)


def _padc(w, cols):
    return jnp.pad(w, ((0, 0), (0, cols - w.shape[1])))


def kernel(x, edge_index, edge_attr, batch, We, be, linW, linb, mlpW, mlpb,
           n2W, n2b, W1, b1, W2, b2, W3, b3):
    row = edge_index[0].astype(jnp.int32)
    col = edge_index[1].astype(jnp.int32)

    weX = _padc(We[:DF], HP)
    weE = _padc(We[DF:], HP)
    beP = _padc(be[None, :], HP)
    linWP = jnp.pad(linW, ((0, 0), (0, HP - H), (0, HP - H)))
    linbP = jnp.pad(linb, ((0, 0), (0, HP - H)))
    mlpWP = jnp.pad(mlpW, ((0, 0), (0, HP - H), (0, HP - H)))
    mlpbP = jnp.pad(mlpb, ((0, 0), (0, HP - H)))
    n2WP = jnp.pad(n2W, ((0, HP - H), (0, HP - H)))
    n2bP = _padc(n2b[None, :], HP)
    w1P = jnp.pad(W1, ((0, HP - H), (0, 0)))

    xw = _tc_matmul(x, weX, jnp.zeros((1, HP), jnp.float32),
                    lambda v: v, 2000)
    xg = _sc_gather(xw, row)
    e = _tc_edge_init(xg, edge_attr, weE, beP, 1000)

    relu = lambda v: jnp.maximum(v, 0.0)
    for l in range(DEPTH):
        msg = _tc_matmul(e, linWP[l], linbP[l][None, :], relu, 1000)
        a = _tc_combine(_sc_scatter_add(msg, col))
        ag = _sc_gather(a, row)
        e = _tc_edge_update(
            ag.reshape(E // 2, 2 * HP), e.reshape(E // 2, 2 * HP),
            mlpWP[l], mlpbP[l][None, :], l == DEPTH - 1, 800,
        ).reshape(E, HP)

    msg_f = _tc_matmul(e, n2WP, n2bP, relu, 1000)
    node_p = _sc_scatter_add(msg_f, col)

    batch2 = batch.astype(jnp.int32)[:, None]
    return _tc_pool_ffn(node_p, batch2, w1P, b1[None, :], W2, b2[None, :],
                        W3, b3[None, :], 2000)


# speedup vs baseline: 3.2456x; 1.0350x over previous
import functools

import jax
import jax.numpy as jnp
from jax import lax
from jax.experimental import pallas as pl
from jax.experimental.pallas import tpu as pltpu
from jax.experimental.pallas import tpu_sc as plsc

N = 10000
E = 160000
DF = 128
DE = 16
H = 300
HP = 384
G = 64
DEPTH = 3

_NC = 2
_NS = 16
_NW = _NC * _NS


def _sc_gather(table, idx):
    e_tot = idx.shape[0]
    ch = 128
    tot_ch = e_tot // ch
    n_ch = tot_ch // _NW
    n_extra = tot_ch - n_ch * _NW
    mesh = plsc.VectorSubcoreMesh(core_axis_name="c", subcore_axis_name="s")

    @functools.partial(
        pl.kernel,
        out_type=jax.ShapeDtypeStruct((e_tot, HP), jnp.float32),
        mesh=mesh,
        scratch_types=[
            pltpu.VMEM((2, ch), jnp.int32),
            pltpu.VMEM((2, ch, HP), jnp.float32),
        ] + [pltpu.SemaphoreType.DMA] * 6,
    )
    def k(table_hbm, idx_hbm, out_hbm, idx_v, rows_v, *sems):
        ixs, gs, ws = sems[0:2], sems[2:4], sems[4:6]
        wid = lax.axis_index("s") * _NC + lax.axis_index("c")

        def base(i):
            return (wid + _NW * i) * ch

        def start_idx(i):
            return pltpu.async_copy(
                idx_hbm.at[pl.ds(base(i), ch)], idx_v.at[i % 2], ixs[i % 2])

        def start_gather(i):
            return pltpu.async_copy(
                table_hbm.at[idx_v.at[i % 2]], rows_v.at[i % 2], gs[i % 2])

        def start_write(i):
            return pltpu.async_copy(
                rows_v.at[i % 2], out_hbm.at[pl.ds(base(i), ch)], ws[i % 2])

        ih, gh, wh = {}, {}, {}
        ih[0] = start_idx(0)
        if n_ch > 1:
            ih[1] = start_idx(1)
        ih[0].wait()
        gh[0] = start_gather(0)
        for i in range(n_ch):
            gh[i].wait()
            wh[i] = start_write(i)
            if i + 2 < n_ch:
                ih[i + 2] = start_idx(i + 2)
            if i + 1 < n_ch:
                if i - 1 >= 0:
                    wh[i - 1].wait()
                ih[i + 1].wait()
                gh[i + 1] = start_gather(i + 1)
        if n_ch > 1:
            wh[n_ch - 2].wait()
        wh[n_ch - 1].wait()

        @pl.when(wid < n_extra)
        def _():
            i = n_ch
            pltpu.async_copy(
                idx_hbm.at[pl.ds(base(i), ch)], idx_v.at[0], ixs[0]).wait()
            pltpu.async_copy(
                table_hbm.at[idx_v.at[0]], rows_v.at[0], gs[0]).wait()
            pltpu.async_copy(
                rows_v.at[0], out_hbm.at[pl.ds(base(i), ch)], ws[0]).wait()

    return k(table, idx)


def _sc_scatter_add(msg, idx):
    e_tot = msg.shape[0]
    per_c = e_tot // _NC
    ch = 80
    tot_ch = per_c // ch
    n_ch = tot_ch // _NS
    n_extra = tot_ch - n_ch * _NS
    slab_rows = 10240
    rows_t = slab_rows // _NS
    zch = 16
    n_phase = HP // 128
    nslot = 4
    mesh = plsc.VectorSubcoreMesh(core_axis_name="c", subcore_axis_name="s")

    @functools.partial(
        pl.kernel,
        out_type=jax.ShapeDtypeStruct((_NC, N, HP), jnp.float32),
        mesh=mesh,
        scratch_types=[
            pltpu.VMEM_SHARED((slab_rows, 128), jnp.float32),
            pltpu.VMEM((nslot, ch), jnp.int32),
            pltpu.VMEM((nslot, ch, 128), jnp.float32),
            pltpu.VMEM((zch, 128), jnp.float32),
        ] + [pltpu.SemaphoreType.DMA] * (3 * nslot),
    )
    def k(msg_hbm, idx_hbm, out_hbm, slab, idx_v, buf_v, zero_v, *sems):
        ixs, ms, sas = (sems[0:nslot], sems[nslot:2 * nslot],
                        sems[2 * nslot:3 * nslot])
        c = lax.axis_index("c")
        s = lax.axis_index("s")

        def base(i):
            return c * per_c + (s + _NS * i) * ch

        zval = jnp.zeros((16,), jnp.float32)
        for zr in range(zch):
            for zc in range(8):
                zero_v[zr, pl.ds(zc * 16, 16)] = zval

        zsem = sas[0]

        def zero_fire_drain():
            hs = []
            for i in range(rows_t // zch):
                hs.append(pltpu.async_copy(
                    zero_v,
                    slab.at[pl.ds(s * rows_t + i * zch, zch)], zsem))
            for h in hs:
                h.wait()

        zero_fire_drain()
        plsc.subcore_barrier()

        for j in range(n_phase):
            def start_loads(i):
                p = i % nslot
                a = pltpu.async_copy(
                    idx_hbm.at[pl.ds(base(i), ch)], idx_v.at[p], ixs[p])
                b = pltpu.async_copy(
                    msg_hbm.at[pl.ds(base(i), ch), pl.ds(j * 128, 128)],
                    buf_v.at[p], ms[p])
                return a, b

            def start_sa(i):
                p = i % nslot
                return pltpu.async_copy(buf_v.at[p], slab.at[idx_v.at[p]],
                                        sas[p], add=True)

            lh, sh = {}, {}
            waited = set()
            for i in range(min(nslot - 1, n_ch)):
                lh[i] = start_loads(i)
            for i in range(n_ch):
                lh[i][0].wait()
                lh[i][1].wait()
                sh[i] = start_sa(i)
                if i + nslot - 1 < n_ch:
                    if i - 1 >= 0:
                        sh[i - 1].wait()
                        waited.add(i - 1)
                    lh[i + nslot - 1] = start_loads(i + nslot - 1)
            for i in range(n_ch):
                if i not in waited:
                    sh[i].wait()

            @pl.when(s < n_extra)
            def _():
                i = n_ch
                a, b = start_loads(i)
                a.wait()
                b.wait()
                start_sa(i).wait()

            plsc.subcore_barrier()

            wr = 640
            nfull = N // wr
            rem = N - nfull * wr

            @pl.when(s < nfull)
            def _():
                pltpu.sync_copy(
                    slab.at[pl.ds(s * wr, wr)],
                    out_hbm.at[c, pl.ds(s * wr, wr), pl.ds(j * 128, 128)])

            @pl.when(s == _NS - 1)
            def _():
                pltpu.sync_copy(
                    slab.at[pl.ds(nfull * wr, rem)],
                    out_hbm.at[c, pl.ds(nfull * wr, rem), pl.ds(j * 128, 128)])

            if j < n_phase - 1:
                zero_fire_drain()
            plsc.subcore_barrier()

    return k(msg, idx)


def _tc_matmul(xin, w, b, act, bm):
    m, kdim = xin.shape
    np_ = w.shape[1]
    grid = m // bm

    def body(x_ref, w_ref, b_ref, o_ref):
        r = jnp.dot(x_ref[...], w_ref[...], preferred_element_type=jnp.float32)
        r = r + b_ref[...]
        o_ref[...] = act(r)

    return pl.pallas_call(
        body,
        grid=(grid,),
        in_specs=[
            pl.BlockSpec((bm, kdim), lambda i: (i, 0)),
            pl.BlockSpec((kdim, np_), lambda i: (0, 0)),
            pl.BlockSpec((1, np_), lambda i: (0, 0)),
        ],
        out_specs=pl.BlockSpec((bm, np_), lambda i: (i, 0)),
        out_shape=jax.ShapeDtypeStruct((m, np_), jnp.float32),
    )(xin, w, b)


def _tc_edge_init(xg, ea, we2, be, bm):
    m = xg.shape[0]
    grid = m // bm

    def body(xg_ref, ea_ref, w_ref, b_ref, o_ref):
        r = jnp.dot(ea_ref[...], w_ref[...], preferred_element_type=jnp.float32)
        r = r + xg_ref[...] + b_ref[...]
        o_ref[...] = jax.nn.silu(r)

    return pl.pallas_call(
        body,
        grid=(grid,),
        in_specs=[
            pl.BlockSpec((bm, HP), lambda i: (i, 0)),
            pl.BlockSpec((bm, DE), lambda i: (i, 0)),
            pl.BlockSpec((DE, HP), lambda i: (0, 0)),
            pl.BlockSpec((1, HP), lambda i: (0, 0)),
        ],
        out_specs=pl.BlockSpec((bm, HP), lambda i: (i, 0)),
        out_shape=jax.ShapeDtypeStruct((m, HP), jnp.float32),
    )(xg, ea, we2, be)


def _tc_edge_update(ag2, e2, w, b, wn, bn_, last, bm2):
    e2_tot = ag2.shape[0]
    grid = e2_tot // bm2

    def body(ag_ref, e_ref, w_ref, b_ref, wn_ref, bn_ref, eo_ref, mo_ref):
        u_l = ag_ref[:, :HP] - e_ref[:, HP:]
        u_r = ag_ref[:, HP:] - e_ref[:, :HP]
        h_l = jnp.dot(u_l, w_ref[...], preferred_element_type=jnp.float32)
        h_r = jnp.dot(u_r, w_ref[...], preferred_element_type=jnp.float32)
        h_l = jnp.maximum(h_l + b_ref[...], 0.0)
        h_r = jnp.maximum(h_r + b_ref[...], 0.0)
        if last:
            o_l = h_l + h_l
            o_r = h_r + h_r
        else:
            o_l = jax.nn.silu(h_l) + h_l
            o_r = jax.nn.silu(h_r) + h_r
        eo_ref[:, :HP] = o_l
        eo_ref[:, HP:] = o_r
        m_l = jnp.dot(o_l, wn_ref[...], preferred_element_type=jnp.float32)
        m_r = jnp.dot(o_r, wn_ref[...], preferred_element_type=jnp.float32)
        mo_ref[:, :HP] = jnp.maximum(m_l + bn_ref[...], 0.0)
        mo_ref[:, HP:] = jnp.maximum(m_r + bn_ref[...], 0.0)

    return pl.pallas_call(
        body,
        grid=(grid,),
        in_specs=[
            pl.BlockSpec((bm2, 2 * HP), lambda i: (i, 0)),
            pl.BlockSpec((bm2, 2 * HP), lambda i: (i, 0)),
            pl.BlockSpec((HP, HP), lambda i: (0, 0)),
            pl.BlockSpec((1, HP), lambda i: (0, 0)),
            pl.BlockSpec((HP, HP), lambda i: (0, 0)),
            pl.BlockSpec((1, HP), lambda i: (0, 0)),
        ],
        out_specs=[pl.BlockSpec((bm2, 2 * HP), lambda i: (i, 0)),
                   pl.BlockSpec((bm2, 2 * HP), lambda i: (i, 0))],
        out_shape=[jax.ShapeDtypeStruct((e2_tot, 2 * HP), jnp.float32),
                   jax.ShapeDtypeStruct((e2_tot, 2 * HP), jnp.float32)],
    )(ag2, e2, w, b, wn, bn_)


def _tc_combine(p1, p2):
    bm = 2000
    grid = N // bm

    def body(p1_ref, p2_ref, o_ref):
        o_ref[...] = (p1_ref[0] + p1_ref[1]) + (p2_ref[0] + p2_ref[1])

    return pl.pallas_call(
        body,
        grid=(grid,),
        in_specs=[pl.BlockSpec((2, bm, HP), lambda i: (0, i, 0)),
                  pl.BlockSpec((2, bm, HP), lambda i: (0, i, 0))],
        out_specs=pl.BlockSpec((bm, HP), lambda i: (i, 0)),
        out_shape=jax.ShapeDtypeStruct((N, HP), jnp.float32),
    )(p1, p2)


def _tc_pool_ffn(np1, np2, batch2, w1, b1, w2, b2, w3, b3, bn):
    grid = N // bn

    def body(nh_ref, nh2_ref, bt_ref, w1_ref, b1_ref, w2_ref, b2_ref, w3_ref,
             b3_ref, o_ref, acc):
        pid = pl.program_id(0)

        @pl.when(pid == 0)
        def _():
            acc[...] = jnp.zeros_like(acc)

        oh = (bt_ref[...] == lax.broadcasted_iota(jnp.int32, (bn, G), 1))
        oh = oh.astype(jnp.float32)
        nh = (nh_ref[0] + nh_ref[1]) + (nh2_ref[0] + nh2_ref[1])
        acc[...] += lax.dot_general(
            oh, nh,
            dimension_numbers=(((0,), (0,)), ((), ())),
            preferred_element_type=jnp.float32)

        @pl.when(pid == grid - 1)
        def _():
            p = acc[...]
            h = p @ w1_ref[...] + b1_ref[...]
            h = jax.nn.silu(h)
            h = h @ w2_ref[...] + b2_ref[...]
            h = jax.nn.silu(h)
            o_ref[...] = h @ w3_ref[...] + b3_ref[...]

    return pl.pallas_call(
        body,
        grid=(grid,),
        in_specs=[
            pl.BlockSpec((2, bn, HP), lambda i: (0, i, 0)),
            pl.BlockSpec((2, bn, HP), lambda i: (0, i, 0)),
            pl.BlockSpec((bn, 1), lambda i: (i, 0)),
            pl.BlockSpec((HP, H), lambda i: (0, 0)),
            pl.BlockSpec((1, H), lambda i: (0, 0)),
            pl.BlockSpec((H, H), lambda i: (0, 0)),
            pl.BlockSpec((1, H), lambda i: (0, 0)),
            pl.BlockSpec((H, 1), lambda i: (0, 0)),
            pl.BlockSpec((1, 1), lambda i: (0, 0)),
        ],
        out_specs=pl.BlockSpec((G, 1), lambda i: (0, 0)),
        out_shape=jax.ShapeDtypeStruct((G, 1), jnp.float32),
        scratch_shapes=[pltpu.VMEM((G, HP), jnp.float32)],
    )(np1, np2, batch2, w1, b1, w2, b2, w3, b3)


def _padc(w, cols):
    return jnp.pad(w, ((0, 0), (0, cols - w.shape[1])))


def kernel(x, edge_index, edge_attr, batch, We, be, linW, linb, mlpW, mlpb,
           n2W, n2b, W1, b1, W2, b2, W3, b3):
    row = edge_index[0].astype(jnp.int32)
    col = edge_index[1].astype(jnp.int32)

    weX = _padc(We[:DF], HP)
    weE = _padc(We[DF:], HP)
    beP = _padc(be[None, :], HP)
    linWP = jnp.pad(linW, ((0, 0), (0, HP - H), (0, HP - H)))
    linbP = jnp.pad(linb, ((0, 0), (0, HP - H)))
    mlpWP = jnp.pad(mlpW, ((0, 0), (0, HP - H), (0, HP - H)))
    mlpbP = jnp.pad(mlpb, ((0, 0), (0, HP - H)))
    n2WP = jnp.pad(n2W, ((0, HP - H), (0, HP - H)))
    n2bP = _padc(n2b[None, :], HP)
    w1P = jnp.pad(W1, ((0, HP - H), (0, 0)))

    EH = E // 2
    rows = (row[:EH], row[EH:])
    cols = (col[:EH], col[EH:])
    eattrs = (edge_attr[:EH], edge_attr[EH:])

    xw = _tc_matmul(x, weX, jnp.zeros((1, HP), jnp.float32),
                    lambda v: v, 2000)
    xg = [_sc_gather(xw, r) for r in rows]
    e = [_tc_edge_init(xg[h], eattrs[h], weE, beP, 1000) for h in range(2)]

    relu = lambda v: jnp.maximum(v, 0.0)
    msg = [_tc_matmul(e[h], linWP[0], linbP[0][None, :], relu, 1000)
           for h in range(2)]
    for l in range(DEPTH):
        p = [_sc_scatter_add(msg[h], cols[h]) for h in range(2)]
        a = _tc_combine(p[0], p[1])
        ag = [_sc_gather(a, r) for r in rows]
        wn = linWP[l + 1] if l < DEPTH - 1 else n2WP
        bn_ = linbP[l + 1][None, :] if l < DEPTH - 1 else n2bP
        res = [_tc_edge_update(
                ag[h].reshape(EH // 2, 2 * HP), e[h].reshape(EH // 2, 2 * HP),
                mlpWP[l], mlpbP[l][None, :], wn, bn_, l == DEPTH - 1, 800)
               for h in range(2)]
        e = [res[h][0].reshape(EH, HP) for h in range(2)]
        msg = [res[h][1].reshape(EH, HP) for h in range(2)]

    node_p = [_sc_scatter_add(msg[h], cols[h]) for h in range(2)]

    batch2 = batch.astype(jnp.int32)[:, None]
    return _tc_pool_ffn(node_p[0], node_p[1], batch2, w1P, b1[None, :], W2,
                        b2[None, :], W3, b3[None, :], 2000)


# speedup vs baseline: 3.3471x; 1.0313x over previous
import functools

import jax
import jax.numpy as jnp
from jax import lax
from jax.experimental import pallas as pl
from jax.experimental.pallas import tpu as pltpu
from jax.experimental.pallas import tpu_sc as plsc

N = 10000
E = 160000
DF = 128
DE = 16
H = 300
HP = 384
G = 64
DEPTH = 3

_NC = 2
_NS = 16
_NW = _NC * _NS


def _sc_gather(table, idx):
    e_tot = idx.shape[0]
    d = table.shape[1]
    ch = 128
    tot_ch = e_tot // ch
    n_ch = tot_ch // _NW
    n_extra = tot_ch - n_ch * _NW
    mesh = plsc.VectorSubcoreMesh(core_axis_name="c", subcore_axis_name="s")

    @functools.partial(
        pl.kernel,
        out_type=jax.ShapeDtypeStruct((e_tot, d), jnp.float32),
        mesh=mesh,
        scratch_types=[
            pltpu.VMEM((2, ch), jnp.int32),
            pltpu.VMEM((2, ch, d), jnp.float32),
        ] + [pltpu.SemaphoreType.DMA] * 6,
    )
    def k(table_hbm, idx_hbm, out_hbm, idx_v, rows_v, *sems):
        ixs, gs, ws = sems[0:2], sems[2:4], sems[4:6]
        wid = lax.axis_index("s") * _NC + lax.axis_index("c")

        def base(i):
            return (wid + _NW * i) * ch

        def start_idx(i):
            return pltpu.async_copy(
                idx_hbm.at[pl.ds(base(i), ch)], idx_v.at[i % 2], ixs[i % 2])

        def start_gather(i):
            return pltpu.async_copy(
                table_hbm.at[idx_v.at[i % 2]], rows_v.at[i % 2], gs[i % 2])

        def start_write(i):
            return pltpu.async_copy(
                rows_v.at[i % 2], out_hbm.at[pl.ds(base(i), ch)], ws[i % 2])

        ih, gh, wh = {}, {}, {}
        ih[0] = start_idx(0)
        if n_ch > 1:
            ih[1] = start_idx(1)
        ih[0].wait()
        gh[0] = start_gather(0)
        for i in range(n_ch):
            gh[i].wait()
            wh[i] = start_write(i)
            if i + 2 < n_ch:
                ih[i + 2] = start_idx(i + 2)
            if i + 1 < n_ch:
                if i - 1 >= 0:
                    wh[i - 1].wait()
                ih[i + 1].wait()
                gh[i + 1] = start_gather(i + 1)
        if n_ch > 1:
            wh[n_ch - 2].wait()
        wh[n_ch - 1].wait()

        @pl.when(wid < n_extra)
        def _():
            i = n_ch
            pltpu.async_copy(
                idx_hbm.at[pl.ds(base(i), ch)], idx_v.at[0], ixs[0]).wait()
            pltpu.async_copy(
                table_hbm.at[idx_v.at[0]], rows_v.at[0], gs[0]).wait()
            pltpu.async_copy(
                rows_v.at[0], out_hbm.at[pl.ds(base(i), ch)], ws[0]).wait()

    return k(table, idx)


def _sc_scatter_add(msg, idx):
    e_tot = msg.shape[0]
    per_c = e_tot // _NC
    ch = 80
    tot_ch = per_c // ch
    n_ch = tot_ch // _NS
    n_extra = tot_ch - n_ch * _NS
    slab_rows = 10240
    rows_t = slab_rows // _NS
    zch = 16
    n_phase = HP // 128
    nslot = 4
    mesh = plsc.VectorSubcoreMesh(core_axis_name="c", subcore_axis_name="s")

    @functools.partial(
        pl.kernel,
        out_type=jax.ShapeDtypeStruct((_NC, N, HP), jnp.float32),
        mesh=mesh,
        scratch_types=[
            pltpu.VMEM_SHARED((slab_rows, 128), jnp.float32),
            pltpu.VMEM((nslot, ch), jnp.int32),
            pltpu.VMEM((nslot, ch, 128), jnp.float32),
            pltpu.VMEM((zch, 128), jnp.float32),
        ] + [pltpu.SemaphoreType.DMA] * (3 * nslot),
    )
    def k(msg_hbm, idx_hbm, out_hbm, slab, idx_v, buf_v, zero_v, *sems):
        ixs, ms, sas = (sems[0:nslot], sems[nslot:2 * nslot],
                        sems[2 * nslot:3 * nslot])
        c = lax.axis_index("c")
        s = lax.axis_index("s")

        def base(i):
            return c * per_c + (s + _NS * i) * ch

        zval = jnp.zeros((16,), jnp.float32)
        for zr in range(zch):
            for zc in range(8):
                zero_v[zr, pl.ds(zc * 16, 16)] = zval

        zsem = sas[0]

        def zero_fire_drain():
            hs = []
            for i in range(rows_t // zch):
                hs.append(pltpu.async_copy(
                    zero_v,
                    slab.at[pl.ds(s * rows_t + i * zch, zch)], zsem))
            for h in hs:
                h.wait()

        zero_fire_drain()
        plsc.subcore_barrier()

        for j in range(n_phase):
            def start_loads(i):
                p = i % nslot
                a = pltpu.async_copy(
                    idx_hbm.at[pl.ds(base(i), ch)], idx_v.at[p], ixs[p])
                b = pltpu.async_copy(
                    msg_hbm.at[pl.ds(base(i), ch), pl.ds(j * 128, 128)],
                    buf_v.at[p], ms[p])
                return a, b

            def start_sa(i):
                p = i % nslot
                return pltpu.async_copy(buf_v.at[p], slab.at[idx_v.at[p]],
                                        sas[p], add=True)

            lh, sh = {}, {}
            waited = set()
            for i in range(min(nslot - 1, n_ch)):
                lh[i] = start_loads(i)
            for i in range(n_ch):
                lh[i][0].wait()
                lh[i][1].wait()
                sh[i] = start_sa(i)
                if i + nslot - 1 < n_ch:
                    if i - 1 >= 0:
                        sh[i - 1].wait()
                        waited.add(i - 1)
                    lh[i + nslot - 1] = start_loads(i + nslot - 1)
            for i in range(n_ch):
                if i not in waited:
                    sh[i].wait()

            @pl.when(s < n_extra)
            def _():
                i = n_ch
                a, b = start_loads(i)
                a.wait()
                b.wait()
                start_sa(i).wait()

            plsc.subcore_barrier()

            wr = 640
            nfull = N // wr
            rem = N - nfull * wr

            @pl.when(s < nfull)
            def _():
                pltpu.sync_copy(
                    slab.at[pl.ds(s * wr, wr)],
                    out_hbm.at[c, pl.ds(s * wr, wr), pl.ds(j * 128, 128)])

            @pl.when(s == _NS - 1)
            def _():
                pltpu.sync_copy(
                    slab.at[pl.ds(nfull * wr, rem)],
                    out_hbm.at[c, pl.ds(nfull * wr, rem), pl.ds(j * 128, 128)])

            if j < n_phase - 1:
                zero_fire_drain()
            plsc.subcore_barrier()

    return k(msg, idx)


def _tc_matmul(xin, w, b, act, bm):
    m, kdim = xin.shape
    np_ = w.shape[1]
    grid = m // bm

    def body(x_ref, w_ref, b_ref, o_ref):
        r = jnp.dot(x_ref[...], w_ref[...], preferred_element_type=jnp.float32)
        r = r + b_ref[...]
        o_ref[...] = act(r)

    return pl.pallas_call(
        body,
        grid=(grid,),
        in_specs=[
            pl.BlockSpec((bm, kdim), lambda i: (i, 0)),
            pl.BlockSpec((kdim, np_), lambda i: (0, 0)),
            pl.BlockSpec((1, np_), lambda i: (0, 0)),
        ],
        out_specs=pl.BlockSpec((bm, np_), lambda i: (i, 0)),
        out_shape=jax.ShapeDtypeStruct((m, np_), jnp.float32),
    )(xin, w, b)


def _tc_edge_init(xg, ea, we1, we2, be, bm):
    m = xg.shape[0]
    grid = m // bm

    def body(xg_ref, ea_ref, w1_ref, w2_ref, b_ref, o_ref):
        r = jnp.dot(xg_ref[...], w1_ref[...], preferred_element_type=jnp.float32)
        r = r + jnp.dot(ea_ref[...], w2_ref[...],
                        preferred_element_type=jnp.float32)
        o_ref[...] = jax.nn.silu(r + b_ref[...])

    return pl.pallas_call(
        body,
        grid=(grid,),
        in_specs=[
            pl.BlockSpec((bm, DF), lambda i: (i, 0)),
            pl.BlockSpec((bm, DE), lambda i: (i, 0)),
            pl.BlockSpec((DF, HP), lambda i: (0, 0)),
            pl.BlockSpec((DE, HP), lambda i: (0, 0)),
            pl.BlockSpec((1, HP), lambda i: (0, 0)),
        ],
        out_specs=pl.BlockSpec((bm, HP), lambda i: (i, 0)),
        out_shape=jax.ShapeDtypeStruct((m, HP), jnp.float32),
    )(xg, ea, we1, we2, be)


def _tc_edge_update(ag2, e2, w, b, wn, bn_, last, bm2):
    e2_tot = ag2.shape[0]
    grid = e2_tot // bm2

    def body(ag_ref, e_ref, w_ref, b_ref, wn_ref, bn_ref, eo_ref, mo_ref):
        u_l = ag_ref[:, :HP] - e_ref[:, HP:]
        u_r = ag_ref[:, HP:] - e_ref[:, :HP]
        h_l = jnp.dot(u_l, w_ref[...], preferred_element_type=jnp.float32)
        h_r = jnp.dot(u_r, w_ref[...], preferred_element_type=jnp.float32)
        h_l = jnp.maximum(h_l + b_ref[...], 0.0)
        h_r = jnp.maximum(h_r + b_ref[...], 0.0)
        if last:
            o_l = h_l + h_l
            o_r = h_r + h_r
        else:
            o_l = jax.nn.silu(h_l) + h_l
            o_r = jax.nn.silu(h_r) + h_r
        eo_ref[:, :HP] = o_l
        eo_ref[:, HP:] = o_r
        m_l = jnp.dot(o_l, wn_ref[...], preferred_element_type=jnp.float32)
        m_r = jnp.dot(o_r, wn_ref[...], preferred_element_type=jnp.float32)
        mo_ref[:, :HP] = jnp.maximum(m_l + bn_ref[...], 0.0)
        mo_ref[:, HP:] = jnp.maximum(m_r + bn_ref[...], 0.0)

    return pl.pallas_call(
        body,
        grid=(grid,),
        in_specs=[
            pl.BlockSpec((bm2, 2 * HP), lambda i: (i, 0)),
            pl.BlockSpec((bm2, 2 * HP), lambda i: (i, 0)),
            pl.BlockSpec((HP, HP), lambda i: (0, 0)),
            pl.BlockSpec((1, HP), lambda i: (0, 0)),
            pl.BlockSpec((HP, HP), lambda i: (0, 0)),
            pl.BlockSpec((1, HP), lambda i: (0, 0)),
        ],
        out_specs=[pl.BlockSpec((bm2, 2 * HP), lambda i: (i, 0)),
                   pl.BlockSpec((bm2, 2 * HP), lambda i: (i, 0))],
        out_shape=[jax.ShapeDtypeStruct((e2_tot, 2 * HP), jnp.float32),
                   jax.ShapeDtypeStruct((e2_tot, 2 * HP), jnp.float32)],
    )(ag2, e2, w, b, wn, bn_)


def _tc_combine(p1, p2):
    bm = 2000
    grid = N // bm

    def body(p1_ref, p2_ref, o_ref):
        o_ref[...] = (p1_ref[0] + p1_ref[1]) + (p2_ref[0] + p2_ref[1])

    return pl.pallas_call(
        body,
        grid=(grid,),
        in_specs=[pl.BlockSpec((2, bm, HP), lambda i: (0, i, 0)),
                  pl.BlockSpec((2, bm, HP), lambda i: (0, i, 0))],
        out_specs=pl.BlockSpec((bm, HP), lambda i: (i, 0)),
        out_shape=jax.ShapeDtypeStruct((N, HP), jnp.float32),
    )(p1, p2)


def _tc_pool_ffn(np1, np2, batch2, w1, b1, w2, b2, w3, b3, bn):
    grid = N // bn

    def body(nh_ref, nh2_ref, bt_ref, w1_ref, b1_ref, w2_ref, b2_ref, w3_ref,
             b3_ref, o_ref, acc):
        pid = pl.program_id(0)

        @pl.when(pid == 0)
        def _():
            acc[...] = jnp.zeros_like(acc)

        oh = (bt_ref[...] == lax.broadcasted_iota(jnp.int32, (bn, G), 1))
        oh = oh.astype(jnp.float32)
        nh = (nh_ref[0] + nh_ref[1]) + (nh2_ref[0] + nh2_ref[1])
        acc[...] += lax.dot_general(
            oh, nh,
            dimension_numbers=(((0,), (0,)), ((), ())),
            preferred_element_type=jnp.float32)

        @pl.when(pid == grid - 1)
        def _():
            p = acc[...]
            h = p @ w1_ref[...] + b1_ref[...]
            h = jax.nn.silu(h)
            h = h @ w2_ref[...] + b2_ref[...]
            h = jax.nn.silu(h)
            o_ref[...] = h @ w3_ref[...] + b3_ref[...]

    return pl.pallas_call(
        body,
        grid=(grid,),
        in_specs=[
            pl.BlockSpec((2, bn, HP), lambda i: (0, i, 0)),
            pl.BlockSpec((2, bn, HP), lambda i: (0, i, 0)),
            pl.BlockSpec((bn, 1), lambda i: (i, 0)),
            pl.BlockSpec((HP, H), lambda i: (0, 0)),
            pl.BlockSpec((1, H), lambda i: (0, 0)),
            pl.BlockSpec((H, H), lambda i: (0, 0)),
            pl.BlockSpec((1, H), lambda i: (0, 0)),
            pl.BlockSpec((H, 1), lambda i: (0, 0)),
            pl.BlockSpec((1, 1), lambda i: (0, 0)),
        ],
        out_specs=pl.BlockSpec((G, 1), lambda i: (0, 0)),
        out_shape=jax.ShapeDtypeStruct((G, 1), jnp.float32),
        scratch_shapes=[pltpu.VMEM((G, HP), jnp.float32)],
    )(np1, np2, batch2, w1, b1, w2, b2, w3, b3)


def _padc(w, cols):
    return jnp.pad(w, ((0, 0), (0, cols - w.shape[1])))


def kernel(x, edge_index, edge_attr, batch, We, be, linW, linb, mlpW, mlpb,
           n2W, n2b, W1, b1, W2, b2, W3, b3):
    row = edge_index[0].astype(jnp.int32)
    col = edge_index[1].astype(jnp.int32)

    weX = _padc(We[:DF], HP)
    weE = _padc(We[DF:], HP)
    beP = _padc(be[None, :], HP)
    linWP = jnp.pad(linW, ((0, 0), (0, HP - H), (0, HP - H)))
    linbP = jnp.pad(linb, ((0, 0), (0, HP - H)))
    mlpWP = jnp.pad(mlpW, ((0, 0), (0, HP - H), (0, HP - H)))
    mlpbP = jnp.pad(mlpb, ((0, 0), (0, HP - H)))
    n2WP = jnp.pad(n2W, ((0, HP - H), (0, HP - H)))
    n2bP = _padc(n2b[None, :], HP)
    w1P = jnp.pad(W1, ((0, HP - H), (0, 0)))

    EH = E // 2
    rows = (row[:EH], row[EH:])
    cols = (col[:EH], col[EH:])
    eattrs = (edge_attr[:EH], edge_attr[EH:])

    xg = [_sc_gather(x, r) for r in rows]
    e = [_tc_edge_init(xg[h], eattrs[h], weX, weE, beP, 1000)
         for h in range(2)]

    relu = lambda v: jnp.maximum(v, 0.0)
    msg = [_tc_matmul(e[h], linWP[0], linbP[0][None, :], relu, 1000)
           for h in range(2)]
    for l in range(DEPTH):
        p = [_sc_scatter_add(msg[h], cols[h]) for h in range(2)]
        a = _tc_combine(p[0], p[1])
        ag = [_sc_gather(a, r) for r in rows]
        wn = linWP[l + 1] if l < DEPTH - 1 else n2WP
        bn_ = linbP[l + 1][None, :] if l < DEPTH - 1 else n2bP
        res = [_tc_edge_update(
                ag[h].reshape(EH // 2, 2 * HP), e[h].reshape(EH // 2, 2 * HP),
                mlpWP[l], mlpbP[l][None, :], wn, bn_, l == DEPTH - 1, 800)
               for h in range(2)]
        e = [res[h][0].reshape(EH, HP) for h in range(2)]
        msg = [res[h][1].reshape(EH, HP) for h in range(2)]

    node_p = [_sc_scatter_add(msg[h], cols[h]) for h in range(2)]

    batch2 = batch.astype(jnp.int32)[:, None]
    return _tc_pool_ffn(node_p[0], node_p[1], batch2, w1P, b1[None, :], W2,
                        b2[None, :], W3, b3[None, :], 2000)


# speedup vs baseline: 3.4357x; 1.0265x over previous
import functools

import jax
import jax.numpy as jnp
from jax import lax
from jax.experimental import pallas as pl
from jax.experimental.pallas import tpu as pltpu
from jax.experimental.pallas import tpu_sc as plsc

N = 10000
E = 160000
DF = 128
DE = 16
H = 300
HP = 384
G = 64
DEPTH = 3

_NC = 2
_NS = 16
_NW = _NC * _NS


def _sc_gather(table, idx):
    e_tot = idx.shape[0]
    d = table.shape[1]
    ch = 128
    tot_ch = e_tot // ch
    n_ch = tot_ch // _NW
    n_extra = tot_ch - n_ch * _NW
    mesh = plsc.VectorSubcoreMesh(core_axis_name="c", subcore_axis_name="s")

    @functools.partial(
        pl.kernel,
        out_type=jax.ShapeDtypeStruct((e_tot, d), jnp.float32),
        mesh=mesh,
        scratch_types=[
            pltpu.VMEM((2, ch), jnp.int32),
            pltpu.VMEM((2, ch, d), jnp.float32),
        ] + [pltpu.SemaphoreType.DMA] * 6,
    )
    def k(table_hbm, idx_hbm, out_hbm, idx_v, rows_v, *sems):
        ixs, gs, ws = sems[0:2], sems[2:4], sems[4:6]
        wid = lax.axis_index("s") * _NC + lax.axis_index("c")

        def base(i):
            return (wid + _NW * i) * ch

        def start_idx(i):
            return pltpu.async_copy(
                idx_hbm.at[pl.ds(base(i), ch)], idx_v.at[i % 2], ixs[i % 2])

        def start_gather(i):
            return pltpu.async_copy(
                table_hbm.at[idx_v.at[i % 2]], rows_v.at[i % 2], gs[i % 2])

        def start_write(i):
            return pltpu.async_copy(
                rows_v.at[i % 2], out_hbm.at[pl.ds(base(i), ch)], ws[i % 2])

        ih, gh, wh = {}, {}, {}
        ih[0] = start_idx(0)
        if n_ch > 1:
            ih[1] = start_idx(1)
        ih[0].wait()
        gh[0] = start_gather(0)
        for i in range(n_ch):
            gh[i].wait()
            wh[i] = start_write(i)
            if i + 2 < n_ch:
                ih[i + 2] = start_idx(i + 2)
            if i + 1 < n_ch:
                if i - 1 >= 0:
                    wh[i - 1].wait()
                ih[i + 1].wait()
                gh[i + 1] = start_gather(i + 1)
        if n_ch > 1:
            wh[n_ch - 2].wait()
        wh[n_ch - 1].wait()

        @pl.when(wid < n_extra)
        def _():
            i = n_ch
            pltpu.async_copy(
                idx_hbm.at[pl.ds(base(i), ch)], idx_v.at[0], ixs[0]).wait()
            pltpu.async_copy(
                table_hbm.at[idx_v.at[0]], rows_v.at[0], gs[0]).wait()
            pltpu.async_copy(
                rows_v.at[0], out_hbm.at[pl.ds(base(i), ch)], ws[0]).wait()

    return k(table, idx)


def _sc_scatter_add(msg, idx):
    e_tot = msg.shape[0]
    per_c = e_tot // _NC
    ch = 80
    tot_ch = per_c // ch
    n_ch = tot_ch // _NS
    n_extra = tot_ch - n_ch * _NS
    slab_rows = 10240
    rows_t = slab_rows // _NS
    zch = 16
    n_phase = HP // 128
    nslot = 4
    mesh = plsc.VectorSubcoreMesh(core_axis_name="c", subcore_axis_name="s")

    @functools.partial(
        pl.kernel,
        out_type=jax.ShapeDtypeStruct((_NC, N, HP), jnp.float32),
        mesh=mesh,
        scratch_types=[
            pltpu.VMEM_SHARED((slab_rows, 128), jnp.float32),
            pltpu.VMEM((nslot, ch), jnp.int32),
            pltpu.VMEM((nslot, ch, 128), jnp.float32),
            pltpu.VMEM((zch, 128), jnp.float32),
        ] + [pltpu.SemaphoreType.DMA] * (3 * nslot),
    )
    def k(msg_hbm, idx_hbm, out_hbm, slab, idx_v, buf_v, zero_v, *sems):
        ixs, ms, sas = (sems[0:nslot], sems[nslot:2 * nslot],
                        sems[2 * nslot:3 * nslot])
        c = lax.axis_index("c")
        s = lax.axis_index("s")

        def base(i):
            return c * per_c + (s + _NS * i) * ch

        zval = jnp.zeros((16,), jnp.float32)
        for zr in range(zch):
            for zc in range(8):
                zero_v[zr, pl.ds(zc * 16, 16)] = zval

        zsem = sas[0]

        def zero_fire_drain():
            hs = []
            for i in range(rows_t // zch):
                hs.append(pltpu.async_copy(
                    zero_v,
                    slab.at[pl.ds(s * rows_t + i * zch, zch)], zsem))
            for h in hs:
                h.wait()

        zero_fire_drain()
        plsc.subcore_barrier()

        for j in range(n_phase):
            def start_loads(i):
                p = i % nslot
                a = pltpu.async_copy(
                    idx_hbm.at[pl.ds(base(i), ch)], idx_v.at[p], ixs[p])
                b = pltpu.async_copy(
                    msg_hbm.at[pl.ds(base(i), ch), pl.ds(j * 128, 128)],
                    buf_v.at[p], ms[p])
                return a, b

            def start_sa(i):
                p = i % nslot
                return pltpu.async_copy(buf_v.at[p], slab.at[idx_v.at[p]],
                                        sas[p], add=True)

            lh, sh = {}, {}
            waited = set()
            for i in range(min(nslot - 1, n_ch)):
                lh[i] = start_loads(i)
            for i in range(n_ch):
                lh[i][0].wait()
                lh[i][1].wait()
                sh[i] = start_sa(i)
                if i + nslot - 1 < n_ch:
                    if i - 1 >= 0:
                        sh[i - 1].wait()
                        waited.add(i - 1)
                    lh[i + nslot - 1] = start_loads(i + nslot - 1)
            for i in range(n_ch):
                if i not in waited:
                    sh[i].wait()

            @pl.when(s < n_extra)
            def _():
                i = n_ch
                a, b = start_loads(i)
                a.wait()
                b.wait()
                start_sa(i).wait()

            plsc.subcore_barrier()

            wr = 640
            nfull = N // wr
            rem = N - nfull * wr

            @pl.when(s < nfull)
            def _():
                pltpu.sync_copy(
                    slab.at[pl.ds(s * wr, wr)],
                    out_hbm.at[c, pl.ds(s * wr, wr), pl.ds(j * 128, 128)])

            @pl.when(s == _NS - 1)
            def _():
                pltpu.sync_copy(
                    slab.at[pl.ds(nfull * wr, rem)],
                    out_hbm.at[c, pl.ds(nfull * wr, rem), pl.ds(j * 128, 128)])

            if j < n_phase - 1:
                zero_fire_drain()
            plsc.subcore_barrier()

    return k(msg, idx)


def _tc_matmul(xin, w, b, act, bm):
    m, kdim = xin.shape
    np_ = w.shape[1]
    grid = m // bm

    def body(x_ref, w_ref, b_ref, o_ref):
        r = jnp.dot(x_ref[...], w_ref[...], preferred_element_type=jnp.float32)
        r = r + b_ref[...]
        o_ref[...] = act(r)

    return pl.pallas_call(
        body,
        grid=(grid,),
        in_specs=[
            pl.BlockSpec((bm, kdim), lambda i: (i, 0)),
            pl.BlockSpec((kdim, np_), lambda i: (0, 0)),
            pl.BlockSpec((1, np_), lambda i: (0, 0)),
        ],
        out_specs=pl.BlockSpec((bm, np_), lambda i: (i, 0)),
        out_shape=jax.ShapeDtypeStruct((m, np_), jnp.float32),
    )(xin, w, b)


def _tc_edge_init(xg, ea, we1, we2, be, bm):
    m = xg.shape[0]
    grid = m // bm

    def body(xg_ref, ea_ref, w1_ref, w2_ref, b_ref, o_ref):
        r = jnp.dot(xg_ref[...], w1_ref[...], preferred_element_type=jnp.float32)
        r = r + jnp.dot(ea_ref[...], w2_ref[...],
                        preferred_element_type=jnp.float32)
        o_ref[...] = jax.nn.silu(r + b_ref[...])

    return pl.pallas_call(
        body,
        grid=(grid,),
        in_specs=[
            pl.BlockSpec((bm, DF), lambda i: (i, 0)),
            pl.BlockSpec((bm, DE), lambda i: (i, 0)),
            pl.BlockSpec((DF, HP), lambda i: (0, 0)),
            pl.BlockSpec((DE, HP), lambda i: (0, 0)),
            pl.BlockSpec((1, HP), lambda i: (0, 0)),
        ],
        out_specs=pl.BlockSpec((bm, HP), lambda i: (i, 0)),
        out_shape=jax.ShapeDtypeStruct((m, HP), jnp.float32),
    )(xg, ea, we1, we2, be)


def _tc_edge_update(ag2, e2, w, b, wn, bn_, last, bm2):
    e2_tot = ag2.shape[0]
    grid = e2_tot // bm2

    def body(ag_ref, e_ref, w_ref, b_ref, wn_ref, bn_ref, eo_ref, mo_ref):
        u_l = ag_ref[:, :HP] - e_ref[:, HP:]
        u_r = ag_ref[:, HP:] - e_ref[:, :HP]
        h_l = jnp.dot(u_l, w_ref[...], preferred_element_type=jnp.float32)
        h_r = jnp.dot(u_r, w_ref[...], preferred_element_type=jnp.float32)
        h_l = jnp.maximum(h_l + b_ref[...], 0.0)
        h_r = jnp.maximum(h_r + b_ref[...], 0.0)
        if last:
            o_l = h_l + h_l
            o_r = h_r + h_r
        else:
            o_l = jax.nn.silu(h_l) + h_l
            o_r = jax.nn.silu(h_r) + h_r
        eo_ref[:, :HP] = o_l
        eo_ref[:, HP:] = o_r
        m_l = jnp.dot(o_l, wn_ref[...], preferred_element_type=jnp.float32)
        m_r = jnp.dot(o_r, wn_ref[...], preferred_element_type=jnp.float32)
        mo_ref[:, :HP] = jnp.maximum(m_l + bn_ref[...], 0.0)
        mo_ref[:, HP:] = jnp.maximum(m_r + bn_ref[...], 0.0)

    return pl.pallas_call(
        body,
        grid=(grid,),
        in_specs=[
            pl.BlockSpec((bm2, 2 * HP), lambda i: (i, 0)),
            pl.BlockSpec((bm2, 2 * HP), lambda i: (i, 0)),
            pl.BlockSpec((HP, HP), lambda i: (0, 0)),
            pl.BlockSpec((1, HP), lambda i: (0, 0)),
            pl.BlockSpec((HP, HP), lambda i: (0, 0)),
            pl.BlockSpec((1, HP), lambda i: (0, 0)),
        ],
        out_specs=[pl.BlockSpec((bm2, 2 * HP), lambda i: (i, 0)),
                   pl.BlockSpec((bm2, 2 * HP), lambda i: (i, 0))],
        out_shape=[jax.ShapeDtypeStruct((e2_tot, 2 * HP), jnp.float32),
                   jax.ShapeDtypeStruct((e2_tot, 2 * HP), jnp.float32)],
    )(ag2, e2, w, b, wn, bn_)


def _tc_combine(p1, p2):
    bm = 2000
    grid = N // bm

    def body(p1_ref, p2_ref, o_ref):
        o_ref[...] = (p1_ref[0] + p1_ref[1]) + (p2_ref[0] + p2_ref[1])

    return pl.pallas_call(
        body,
        grid=(grid,),
        in_specs=[pl.BlockSpec((2, bm, HP), lambda i: (0, i, 0)),
                  pl.BlockSpec((2, bm, HP), lambda i: (0, i, 0))],
        out_specs=pl.BlockSpec((bm, HP), lambda i: (i, 0)),
        out_shape=jax.ShapeDtypeStruct((N, HP), jnp.float32),
    )(p1, p2)


def _tc_pool_ffn(np1, np2, batch2, w1, b1, w2, b2, w3, b3, bn):
    grid = N // bn

    def body(nh_ref, nh2_ref, bt_ref, w1_ref, b1_ref, w2_ref, b2_ref, w3_ref,
             b3_ref, o_ref, acc):
        pid = pl.program_id(0)

        @pl.when(pid == 0)
        def _():
            acc[...] = jnp.zeros_like(acc)

        oh = (bt_ref[...] == lax.broadcasted_iota(jnp.int32, (bn, G), 1))
        oh = oh.astype(jnp.float32)
        nh = (nh_ref[0] + nh_ref[1]) + (nh2_ref[0] + nh2_ref[1])
        acc[...] += lax.dot_general(
            oh, nh,
            dimension_numbers=(((0,), (0,)), ((), ())),
            preferred_element_type=jnp.float32)

        @pl.when(pid == grid - 1)
        def _():
            p = acc[...]
            h = p @ w1_ref[...] + b1_ref[...]
            h = jax.nn.silu(h)
            h = h @ w2_ref[...] + b2_ref[...]
            h = jax.nn.silu(h)
            o_ref[...] = h @ w3_ref[...] + b3_ref[...]

    return pl.pallas_call(
        body,
        grid=(grid,),
        in_specs=[
            pl.BlockSpec((2, bn, HP), lambda i: (0, i, 0)),
            pl.BlockSpec((2, bn, HP), lambda i: (0, i, 0)),
            pl.BlockSpec((bn, 1), lambda i: (i, 0)),
            pl.BlockSpec((HP, H), lambda i: (0, 0)),
            pl.BlockSpec((1, H), lambda i: (0, 0)),
            pl.BlockSpec((H, H), lambda i: (0, 0)),
            pl.BlockSpec((1, H), lambda i: (0, 0)),
            pl.BlockSpec((H, 1), lambda i: (0, 0)),
            pl.BlockSpec((1, 1), lambda i: (0, 0)),
        ],
        out_specs=pl.BlockSpec((G, 1), lambda i: (0, 0)),
        out_shape=jax.ShapeDtypeStruct((G, 1), jnp.float32),
        scratch_shapes=[pltpu.VMEM((G, HP), jnp.float32)],
    )(np1, np2, batch2, w1, b1, w2, b2, w3, b3)


def _padc(w, cols):
    return jnp.pad(w, ((0, 0), (0, cols - w.shape[1])))


def kernel(x, edge_index, edge_attr, batch, We, be, linW, linb, mlpW, mlpb,
           n2W, n2b, W1, b1, W2, b2, W3, b3):
    row = edge_index[0].astype(jnp.int32)
    col = edge_index[1].astype(jnp.int32)

    weX = _padc(We[:DF], HP)
    weE = _padc(We[DF:], HP)
    beP = _padc(be[None, :], HP)
    linWP = jnp.pad(linW, ((0, 0), (0, HP - H), (0, HP - H)))
    linbP = jnp.pad(linb, ((0, 0), (0, HP - H)))
    mlpWP = jnp.pad(mlpW, ((0, 0), (0, HP - H), (0, HP - H)))
    mlpbP = jnp.pad(mlpb, ((0, 0), (0, HP - H)))
    n2WP = jnp.pad(n2W, ((0, HP - H), (0, HP - H)))
    n2bP = _padc(n2b[None, :], HP)
    w1P = jnp.pad(W1, ((0, HP - H), (0, 0)))

    EH = E // 2
    rows = (row[:EH], row[EH:])
    cols = (col[:EH], col[EH:])
    eattrs = (edge_attr[:EH], edge_attr[EH:])

    xg = [_sc_gather(x, r) for r in rows]
    e = [_tc_edge_init(xg[h], eattrs[h], weX, weE, beP, 2000)
         for h in range(2)]

    relu = lambda v: jnp.maximum(v, 0.0)
    msg = [_tc_matmul(e[h], linWP[0], linbP[0][None, :], relu, 2000)
           for h in range(2)]
    for l in range(DEPTH):
        p = [_sc_scatter_add(msg[h], cols[h]) for h in range(2)]
        a = _tc_combine(p[0], p[1])
        ag = [_sc_gather(a, r) for r in rows]
        wn = linWP[l + 1] if l < DEPTH - 1 else n2WP
        bn_ = linbP[l + 1][None, :] if l < DEPTH - 1 else n2bP
        res = [_tc_edge_update(
                ag[h].reshape(EH // 2, 2 * HP), e[h].reshape(EH // 2, 2 * HP),
                mlpWP[l], mlpbP[l][None, :], wn, bn_, l == DEPTH - 1, 1000)
               for h in range(2)]
        e = [res[h][0].reshape(EH, HP) for h in range(2)]
        msg = [res[h][1].reshape(EH, HP) for h in range(2)]

    node_p = [_sc_scatter_add(msg[h], cols[h]) for h in range(2)]

    batch2 = batch.astype(jnp.int32)[:, None]
    return _tc_pool_ffn(node_p[0], node_p[1], batch2, w1P, b1[None, :], W2,
                        b2[None, :], W3, b3[None, :], 2000)
